# baseline (speedup 1.0000x reference)
import jax
import jax.numpy as jnp
from jax.experimental import pallas as pl

N_ROIS = 268
D_MODEL = 64
BATCH = 4096
BB = 256


def _bcast_kernel(tab_ref, out_ref):
    out_ref[...] = jnp.broadcast_to(tab_ref[...], out_ref.shape)


def kernel(batch_size, pos_embedding):
    tab3 = pos_embedding.reshape(N_ROIS, D_MODEL, 1)
    out = pl.pallas_call(
        _bcast_kernel,
        grid=(BATCH // BB,),
        in_specs=[pl.BlockSpec((N_ROIS, D_MODEL, 1), lambda i: (0, 0, 0))],
        out_specs=pl.BlockSpec((N_ROIS, D_MODEL, BB), lambda i: (0, 0, i)),
        out_shape=jax.ShapeDtypeStruct((N_ROIS, D_MODEL, BATCH), jnp.float32),
    )(tab3)
    return jnp.transpose(out, (2, 0, 1))


# speedup vs baseline: 1.0042x; 1.0042x over previous
import jax
import jax.numpy as jnp
from jax.experimental import pallas as pl
from jax.experimental.pallas import tpu as pltpu

N_ROIS = 268
D_MODEL = 64
BATCH = 4096
BB = 256
STEPS = BATCH // BB


def _bcast_kernel(tab_ref, out_ref, buf, sems):
    buf[...] = jnp.broadcast_to(tab_ref[...], buf.shape)

    def dma(i, slot):
        return pltpu.make_async_copy(
            buf, out_ref.at[:, :, pl.ds(i * BB, BB)], sems.at[slot]
        )

    def body(i, carry):
        dma(i, jax.lax.rem(i, 2)).start()

        @pl.when(i > 0)
        def _():
            dma(i - 1, jax.lax.rem(i - 1, 2)).wait()

        return carry

    jax.lax.fori_loop(0, STEPS, body, 0)
    dma(STEPS - 1, jax.lax.rem(STEPS - 1, 2)).wait()


def kernel(batch_size, pos_embedding):
    tab3 = pos_embedding.reshape(N_ROIS, D_MODEL, 1)
    out = pl.pallas_call(
        _bcast_kernel,
        in_specs=[pl.BlockSpec((N_ROIS, D_MODEL, 1), lambda: (0, 0, 0))],
        out_specs=pl.BlockSpec(memory_space=pltpu.HBM),
        out_shape=jax.ShapeDtypeStruct((N_ROIS, D_MODEL, BATCH), jnp.float32),
        scratch_shapes=[
            pltpu.VMEM((N_ROIS, D_MODEL, BB), jnp.float32),
            pltpu.SemaphoreType.DMA((2,)),
        ],
    )(tab3)
    return jnp.transpose(out, (2, 0, 1))
